# baseline (speedup 1.0000x reference)
import math

import jax
import jax.numpy as jnp
from jax import lax
from jax.experimental import pallas as pl
from jax.experimental.pallas import tpu as pltpu
from jax.experimental.pallas import tpu_sc as plsc

N_TOKENS = 32768
IN_FEATURES = 2048
OUT_FEATURES = 2048
RANK = 16
NUM_EXPERTS = 64
TOP_K = 8
SCALING = 32 / 16

BLK1 = 2048
BLK2 = 2048
_INV_SQRT2 = 1.0 / math.sqrt(2.0)
_NEG_KEY = -2147483648

NC = 2
NS = 16
NW = NC * NS
TOK_PER_W = N_TOKENS // NW
GRP = 16


def _pass1_body(x_ref, raT_ref, sT_ref, xa_ref):
    x = x_ref[...]
    sxa = jnp.dot(x, raT_ref[...], preferred_element_type=jnp.float32)
    sT_ref[...] = sxa.T[:NUM_EXPERTS, :]
    xa_ref[...] = sxa[:, NUM_EXPERTS:NUM_EXPERTS + RANK]


def _pass1(x, raT):
    n = x.shape[0]
    return pl.pallas_call(
        _pass1_body,
        grid=(n // BLK1,),
        in_specs=[
            pl.BlockSpec((BLK1, IN_FEATURES), lambda i: (i, 0)),
            pl.BlockSpec((IN_FEATURES, 128), lambda i: (0, 0)),
        ],
        out_specs=[
            pl.BlockSpec((NUM_EXPERTS, BLK1), lambda i: (0, i)),
            pl.BlockSpec((BLK1, RANK), lambda i: (i, 0)),
        ],
        out_shape=[
            jax.ShapeDtypeStruct((NUM_EXPERTS, n), jnp.float32),
            jax.ShapeDtypeStruct((n, RANK), jnp.float32),
        ],
    )(x, raT)


def _route_body(sT_hbm, ev_hbm, out_hbm, s_v, ev_v, e_v, dma_sem):
    wid = lax.axis_index("s") * NC + lax.axis_index("c")
    base = wid * TOK_PER_W
    pltpu.sync_copy(sT_hbm.at[:, pl.ds(base, TOK_PER_W)], s_v)
    pltpu.sync_copy(ev_hbm, ev_v)

    def group(g, carry):
        slots = [jnp.full((GRP,), -jnp.inf, jnp.float32) for _ in range(TOP_K)]
        ids = [jnp.zeros((GRP,), jnp.int32) for _ in range(TOP_K)]
        for e in range(NUM_EXPERTS):
            key = s_v[e, pl.ds(g * GRP, GRP)]
            kid = jnp.full((GRP,), e, jnp.int32)
            for t in range(TOP_K):
                gt = key > slots[t]
                hi = jnp.where(gt, key, slots[t])
                key = jnp.where(gt, slots[t], key)
                slots[t] = hi
                hid = jnp.where(gt, kid, ids[t])
                kid = jnp.where(gt, ids[t], kid)
                ids[t] = hid
        es = [jnp.exp(slots[t] - slots[0]) for t in range(TOP_K)]
        denom = es[0]
        for t in range(1, TOP_K):
            denom = denom + es[t]
        inv = 1.0 / denom
        eids = ids
        ws = [es[t] * inv for t in range(TOP_K)]
        for t in range(GRP):
            acc = jnp.zeros((GRP,), jnp.float32)
            for j in range(TOP_K):
                eid_s = eids[j][t]
                w_s = ws[j][t]
                acc = acc + w_s * ev_v[pl.ds(eid_s * RANK, RANK)]
            e_v[pl.ds((g * GRP + t) * RANK, RANK)] = acc
        return carry

    lax.fori_loop(0, TOK_PER_W // GRP, group, 0)
    pltpu.sync_copy(e_v, out_hbm.at[pl.ds(base * RANK, TOK_PER_W * RANK)])


def _route(sT, ev):
    mesh = plsc.VectorSubcoreMesh(core_axis_name="c", subcore_axis_name="s")
    run = pl.kernel(
        _route_body,
        mesh=mesh,
        out_type=jax.ShapeDtypeStruct((N_TOKENS * RANK,), jnp.float32),
        scratch_types=[
            pltpu.VMEM((NUM_EXPERTS, TOK_PER_W), jnp.float32),
            pltpu.VMEM((NUM_EXPERTS * RANK,), jnp.float32),
            pltpu.VMEM((TOK_PER_W * RANK,), jnp.float32),
            pltpu.SemaphoreType.DMA,
        ],
    )
    return run(sT, ev.reshape(-1))


def _pass2_body(xa_ref, e_ref, bwT_ref, out_ref):
    h = xa_ref[...].T + e_ref[...].T
    g = 0.5 * h * (1.0 + jax.lax.erf(h * _INV_SQRT2))
    out_ref[...] = jax.lax.dot_general(
        g, bwT_ref[...], (((0,), (0,)), ((), ())),
        preferred_element_type=jnp.float32)


def _pass2(xa, etok, bwT):
    n = xa.shape[0]
    return pl.pallas_call(
        _pass2_body,
        grid=(n // BLK2,),
        in_specs=[
            pl.BlockSpec((BLK2, RANK), lambda i: (i, 0)),
            pl.BlockSpec((BLK2, RANK), lambda i: (i, 0)),
            pl.BlockSpec((RANK, OUT_FEATURES), lambda i: (0, 0)),
        ],
        out_specs=pl.BlockSpec((BLK2, OUT_FEATURES), lambda i: (i, 0)),
        out_shape=jax.ShapeDtypeStruct((n, OUT_FEATURES), jnp.float32),
    )(xa, etok, bwT)


def kernel(x, A_w, B_w, expert_vectors, router_w):
    raT = jnp.zeros((IN_FEATURES, 128), jnp.float32)
    raT = raT.at[:, :NUM_EXPERTS].set(router_w.T)
    raT = raT.at[:, NUM_EXPERTS:NUM_EXPERTS + RANK].set(A_w.T)
    bwT = B_w.T * SCALING
    sT, xa = _pass1(x, raT)
    etok = _route(sT, expert_vectors).reshape(N_TOKENS, RANK)
    return _pass2(xa, etok, bwT)


# speedup vs baseline: 1.3262x; 1.3262x over previous
import math

import jax
import jax.numpy as jnp
from jax.experimental import pallas as pl
from jax.experimental.pallas import tpu as pltpu

N_TOKENS = 32768
IN_FEATURES = 2048
OUT_FEATURES = 2048
RANK = 16
NUM_EXPERTS = 64
TOP_K = 8
SCALING = 32 / 16

BLK = 1024
_INV_SQRT2 = 1.0 / math.sqrt(2.0)
_NEG_KEY = -2147483648


SUB = 256


def _fused_body(x_ref, raT_ref, evT_ref, bwT_ref, out_ref):
    x = x_ref[...]
    sxa = jnp.dot(x, raT_ref[...], preferred_element_type=jnp.float32)
    t = sxa.T

    for c in range(BLK // SUB):
        s = t[:NUM_EXPERTS, c * SUB:(c + 1) * SUB]
        xa = t[NUM_EXPERTS:NUM_EXPERTS + RANK, c * SUB:(c + 1) * SUB]

        row = jax.lax.broadcasted_iota(jnp.int32, s.shape, 0)
        u = jax.lax.bitcast_convert_type(s, jnp.int32)
        key = u ^ ((u >> 31) & jnp.int32(0x7FFFFFFF))
        cur = (key & jnp.int32(~0x3F)) | (jnp.int32(NUM_EXPERTS - 1) - row)

        m1 = jnp.max(s, axis=0, keepdims=True)
        ex = jnp.exp(s - m1)

        for j in range(TOP_K):
            mkey = jnp.max(cur, axis=0, keepdims=True)
            cur = jnp.where(cur == mkey, jnp.int32(_NEG_KEY), cur)

        wnum = jnp.where(cur == jnp.int32(_NEG_KEY), ex, 0.0)
        denom = jnp.sum(wnum, axis=0, keepdims=True)

        etok = jnp.dot(evT_ref[...], wnum, preferred_element_type=jnp.float32)
        h = xa + etok / denom
        g = 0.5 * h * (1.0 + jax.lax.erf(h * _INV_SQRT2))
        out_ref[pl.ds(c * SUB, SUB), :] = jax.lax.dot_general(
            g, bwT_ref[...], (((0,), (0,)), ((), ())),
            preferred_element_type=jnp.float32)


def kernel(x, A_w, B_w, expert_vectors, router_w):
    n = x.shape[0]
    grid = n // BLK
    raT = jnp.zeros((IN_FEATURES, 128), jnp.float32)
    raT = raT.at[:, :NUM_EXPERTS].set(router_w.T)
    raT = raT.at[:, NUM_EXPERTS:NUM_EXPERTS + RANK].set(A_w.T)
    evT = expert_vectors.T
    bwT = B_w.T * SCALING
    return pl.pallas_call(
        _fused_body,
        grid=(grid,),
        in_specs=[
            pl.BlockSpec((BLK, IN_FEATURES), lambda i: (i, 0)),
            pl.BlockSpec((IN_FEATURES, 128), lambda i: (0, 0)),
            pl.BlockSpec((RANK, NUM_EXPERTS), lambda i: (0, 0)),
            pl.BlockSpec((RANK, OUT_FEATURES), lambda i: (0, 0)),
        ],
        out_specs=pl.BlockSpec((BLK, OUT_FEATURES), lambda i: (i, 0)),
        out_shape=jax.ShapeDtypeStruct((n, OUT_FEATURES), jnp.float32),
    )(x, raT, evT, bwT)


# speedup vs baseline: 1.3756x; 1.0372x over previous
import math

import jax
import jax.numpy as jnp
from jax.experimental import pallas as pl
from jax.experimental.pallas import tpu as pltpu

N_TOKENS = 32768
IN_FEATURES = 2048
OUT_FEATURES = 2048
RANK = 16
NUM_EXPERTS = 64
TOP_K = 8
SCALING = 32 / 16

BLK = 1024
_INV_SQRT2 = 1.0 / math.sqrt(2.0)
_NEG_KEY = -2147483648


def _fused_body(x_ref, raT_ref, evT_ref, bwT_ref, out_ref):
    x = x_ref[...]
    sxa = jnp.dot(x, raT_ref[...], preferred_element_type=jnp.float32)
    t = sxa.T
    s = t[:NUM_EXPERTS, :]
    xa = t[NUM_EXPERTS:NUM_EXPERTS + RANK, :]

    row = jax.lax.broadcasted_iota(jnp.int32, s.shape, 0)
    u = jax.lax.bitcast_convert_type(s, jnp.int32)
    key = u ^ ((u >> 31) & jnp.int32(0x7FFFFFFF))
    cur = (key & jnp.int32(~0x3F)) | (jnp.int32(NUM_EXPERTS - 1) - row)

    m1 = jnp.max(s, axis=0, keepdims=True)
    ex = jnp.exp(s - m1)

    for j in range(TOP_K):
        mkey = jnp.max(cur, axis=0, keepdims=True)
        cur = jnp.where(cur == mkey, jnp.int32(_NEG_KEY), cur)

    wnum = jnp.where(cur == jnp.int32(_NEG_KEY), ex, 0.0)
    denom = jnp.sum(wnum, axis=0, keepdims=True)

    etok = jnp.dot(evT_ref[...], wnum, preferred_element_type=jnp.float32)
    h = xa + etok / denom
    g = 0.5 * h * (1.0 + jax.lax.erf(h * _INV_SQRT2))
    out_ref[...] = jax.lax.dot_general(
        g, bwT_ref[...], (((0,), (0,)), ((), ())),
        preferred_element_type=jnp.float32)


def kernel(x, A_w, B_w, expert_vectors, router_w):
    n = x.shape[0]
    grid = n // BLK
    raT = jnp.zeros((IN_FEATURES, 128), jnp.float32)
    raT = raT.at[:, :NUM_EXPERTS].set(router_w.T)
    raT = raT.at[:, NUM_EXPERTS:NUM_EXPERTS + RANK].set(A_w.T)
    evT = expert_vectors.T
    bwT = B_w.T * SCALING
    return pl.pallas_call(
        _fused_body,
        grid=(grid,),
        in_specs=[
            pl.BlockSpec((BLK, IN_FEATURES), lambda i: (i, 0)),
            pl.BlockSpec((IN_FEATURES, 128), lambda i: (0, 0)),
            pl.BlockSpec((RANK, NUM_EXPERTS), lambda i: (0, 0)),
            pl.BlockSpec((RANK, OUT_FEATURES), lambda i: (0, 0)),
        ],
        out_specs=pl.BlockSpec((BLK, OUT_FEATURES), lambda i: (i, 0)),
        compiler_params=pltpu.CompilerParams(dimension_semantics=("parallel",)),
        out_shape=jax.ShapeDtypeStruct((n, OUT_FEATURES), jnp.float32),
    )(x, raT, evT, bwT)
